# Initial kernel scaffold; baseline (speedup 1.0000x reference)
import jax
import jax.numpy as jnp
from jax.experimental import pallas as pl


def kernel(x, poi, emb_poi_weight, emb_loc_weight):
    raise NotImplementedError("write your pallas kernel here")


# speedup vs baseline: 1.2573x; 1.2573x over previous
import jax
import jax.numpy as jnp
from jax import lax
from jax.experimental import pallas as pl
from jax.experimental.pallas import tpu as pltpu
from jax.experimental.pallas import tpu_sc as plsc

LOC_EMB_SIZE = 64
N_POI_CAT = 16
N_VALS = 11
BATCH = 1024
ALPHA = 0.9
L = 16
NC, NS = 2, 16
NW = NC * NS
B_PER_W = BATCH // NW
DGRP = LOC_EMB_SIZE // L
NBLK = B_PER_W // L


def _body(x_hbm, poi_hbm, w_hbm, loc_hbm, out_hbm,
          idx_v, poi_rows_v, loc_rows_v, w_v, e_v, e2_v, cnt_v, out_rows_v,
          sem_poi, sem_loc):
    wid = lax.axis_index("s") * NC + lax.axis_index("c")
    base = wid * B_PER_W

    pltpu.sync_copy(x_hbm.at[pl.ds(base, B_PER_W)], idx_v)
    cp_poi = pltpu.async_copy(poi_hbm.at[idx_v], poi_rows_v, sem_poi)
    cp_loc = pltpu.async_copy(loc_hbm.at[idx_v], loc_rows_v, sem_loc)

    pltpu.sync_copy(w_hbm, w_v)
    for v in range(N_VALS):
        for j in range(DGRP):
            e = jnp.exp(w_v[v, pl.ds(j * L, L)])
            e_v[v, pl.ds(j * L, L)] = e
            e2_v[v, pl.ds(j * L, L)] = e * e

    cp_poi.wait()

    lane = lax.iota(jnp.int32, L)
    onef = jnp.ones((L,), jnp.float32)
    zerof = jnp.zeros((L,), jnp.float32)
    for t in range(NBLK):
        rows = t * L + lane
        cnts = [zerof] * N_VALS
        for k in range(N_POI_CAT):
            col = plsc.load_gather(poi_rows_v, [rows, jnp.full((L,), k, jnp.int32)])
            for v in range(N_VALS):
                ind = jnp.maximum(onef - jnp.abs(col - float(v)), zerof)
                cnts[v] = cnts[v] + ind
        for v in range(N_VALS):
            cnt_v[v, pl.ds(t * L, L)] = cnts[v]

    cp_loc.wait()

    for i in range(B_PER_W):
        cfs = [plsc.load_gather(cnt_v, [jnp.full((L,), v, jnp.int32),
                                        jnp.full((L,), i, jnp.int32)])
               for v in range(N_VALS)]
        for j in range(DGRP):
            den = cfs[0] * e_v[0, pl.ds(j * L, L)]
            num = cfs[0] * e2_v[0, pl.ds(j * L, L)]
            for v in range(1, N_VALS):
                den = den + cfs[v] * e_v[v, pl.ds(j * L, L)]
                num = num + cfs[v] * e2_v[v, pl.ds(j * L, L)]
            p = num / den
            loc = loc_rows_v[i, pl.ds(j * L, L)]
            out_rows_v[i, pl.ds(j * L, L)] = loc * ALPHA + p * (1.0 - ALPHA)

    pltpu.sync_copy(out_rows_v, out_hbm.at[pl.ds(base, B_PER_W)])


@jax.jit
def kernel(x, poi, emb_poi_weight, emb_loc_weight):
    run = pl.kernel(
        _body,
        out_type=jax.ShapeDtypeStruct((BATCH, LOC_EMB_SIZE), jnp.float32),
        mesh=plsc.VectorSubcoreMesh(core_axis_name="c", subcore_axis_name="s"),
        compiler_params=pltpu.CompilerParams(needs_layout_passes=False,
                                             use_tc_tiling_on_sc=False),
        scratch_types=[
            pltpu.VMEM((B_PER_W,), jnp.int32),
            pltpu.VMEM((B_PER_W, N_POI_CAT), jnp.float32),
            pltpu.VMEM((B_PER_W, LOC_EMB_SIZE), jnp.float32),
            pltpu.VMEM((N_VALS, LOC_EMB_SIZE), jnp.float32),
            pltpu.VMEM((N_VALS, LOC_EMB_SIZE), jnp.float32),
            pltpu.VMEM((N_VALS, LOC_EMB_SIZE), jnp.float32),
            pltpu.VMEM((N_VALS, B_PER_W), jnp.float32),
            pltpu.VMEM((B_PER_W, LOC_EMB_SIZE), jnp.float32),
            pltpu.SemaphoreType.DMA,
            pltpu.SemaphoreType.DMA,
        ],
    )
    return run(x.astype(jnp.int32), poi, emb_poi_weight, emb_loc_weight)


# speedup vs baseline: 2.7490x; 2.1865x over previous
import jax
import jax.numpy as jnp
from jax import lax
from jax.experimental import pallas as pl
from jax.experimental.pallas import tpu as pltpu
from jax.experimental.pallas import tpu_sc as plsc

LOC_EMB_SIZE = 64
N_POI_CAT = 16
N_VALS = 11
BATCH = 1024
POINT = 100000
ALPHA = 0.9
L = 16
NC, NS = 2, 16
D_PER_TILE = LOC_EMB_SIZE // (NC * NS)
NGRP = BATCH // L


def _body(x_hbm, poi_t_hbm, w_hbm, loc_t_hbm, out_t_hbm,
          x_v, row_v, idx_v, w_v, e_tab_v, out_rows_v, gath_sh):
    c = lax.axis_index("c")
    s = lax.axis_index("s")

    pltpu.sync_copy(x_hbm, x_v)
    pltpu.sync_copy(w_hbm, w_v)

    pltpu.sync_copy(poi_t_hbm.at[s], row_v)
    for g in range(NGRP):
        xg = x_v[pl.ds(g * L, L)]
        val = plsc.load_gather(row_v, [xg])
        idx_v[s, pl.ds(g * L, L)] = val.astype(jnp.int32)
    pltpu.sync_copy(idx_v.at[s], gath_sh.at[s])
    plsc.subcore_barrier()
    pltpu.sync_copy(gath_sh, idx_v)

    lane = lax.iota(jnp.int32, L)
    vlane = jnp.minimum(lane, N_VALS - 1)
    for dd in range(D_PER_TILE):
        d = (c * NS + s) * D_PER_TILE + dd
        pltpu.sync_copy(loc_t_hbm.at[d], row_v)
        w_col = plsc.load_gather(w_v, [vlane, jnp.broadcast_to(d, (L,))])
        e_vec = jnp.exp(w_col)
        e_tab_v[...] = e_vec
        for g in range(NGRP):
            xg = x_v[pl.ds(g * L, L)]
            lv = plsc.load_gather(row_v, [xg])
            ik = idx_v[0, pl.ds(g * L, L)]
            e = plsc.load_gather(e_tab_v, [ik])
            den = e
            num = e * e
            for k in range(1, N_POI_CAT):
                ik = idx_v[k, pl.ds(g * L, L)]
                e = plsc.load_gather(e_tab_v, [ik])
                den = den + e
                num = num + e * e
            p = num / den
            out_rows_v[dd, pl.ds(g * L, L)] = lv * ALPHA + p * (1.0 - ALPHA)

    d0 = (c * NS + s) * D_PER_TILE
    pltpu.sync_copy(out_rows_v, out_t_hbm.at[pl.ds(d0, D_PER_TILE)])


@jax.jit
def kernel(x, poi, emb_poi_weight, emb_loc_weight):
    run = pl.kernel(
        _body,
        out_type=jax.ShapeDtypeStruct((LOC_EMB_SIZE, BATCH), jnp.float32),
        mesh=plsc.VectorSubcoreMesh(core_axis_name="c", subcore_axis_name="s"),
        compiler_params=pltpu.CompilerParams(needs_layout_passes=False,
                                             use_tc_tiling_on_sc=True),
        scratch_types=[
            pltpu.VMEM((BATCH,), jnp.int32),
            pltpu.VMEM((POINT,), jnp.float32),
            pltpu.VMEM((N_POI_CAT, BATCH), jnp.int32),
            pltpu.VMEM((N_VALS, LOC_EMB_SIZE), jnp.float32),
            pltpu.VMEM((L,), jnp.float32),
            pltpu.VMEM((D_PER_TILE, BATCH), jnp.float32),
            pltpu.VMEM_SHARED((N_POI_CAT, BATCH), jnp.int32),
        ],
    )
    out_t = run(x.astype(jnp.int32), poi.T, emb_poi_weight, emb_loc_weight.T)
    return out_t.T


# speedup vs baseline: 3.6162x; 1.3154x over previous
import jax
import jax.numpy as jnp
from jax import lax
from jax.experimental import pallas as pl
from jax.experimental.pallas import tpu as pltpu
from jax.experimental.pallas import tpu_sc as plsc

LOC_EMB_SIZE = 64
N_POI_CAT = 16
N_VALS = 11
BATCH = 1024
POINT = 100000
ALPHA = 0.9
L = 16
NC, NS = 2, 16
D_PER_TILE = LOC_EMB_SIZE // (NC * NS)
NGRP = BATCH // L


def _body(x_hbm, poi_t_hbm, w_hbm, loc_t_hbm, out_t_hbm,
          x_v, row_v, idx_v, w_v, e_tab_v, out_rows_v, gath_sh):
    c = lax.axis_index("c")
    s = lax.axis_index("s")

    pltpu.sync_copy(x_hbm, x_v)
    pltpu.sync_copy(w_hbm, w_v)

    pltpu.sync_copy(poi_t_hbm.at[s], row_v)

    def _phase_a(g, carry):
        xg = x_v[pl.ds(g * L, L)]
        val = plsc.load_gather(row_v, [xg])
        idx_v[s, pl.ds(g * L, L)] = val.astype(jnp.int32)
        return carry

    lax.fori_loop(0, NGRP, _phase_a, 0, unroll=4)
    pltpu.sync_copy(idx_v.at[s], gath_sh.at[s])
    plsc.subcore_barrier()
    pltpu.sync_copy(gath_sh, idx_v)

    lane = lax.iota(jnp.int32, L)
    vlane = jnp.minimum(lane, N_VALS - 1)
    for dd in range(D_PER_TILE):
        d = (c * NS + s) * D_PER_TILE + dd
        pltpu.sync_copy(loc_t_hbm.at[d], row_v)
        w_col = plsc.load_gather(w_v, [vlane, jnp.broadcast_to(d, (L,))])
        e_vec = jnp.exp(w_col)
        e_tab_v[...] = e_vec
        def _phase_b(g, carry):
            xg = x_v[pl.ds(g * L, L)]
            lv = plsc.load_gather(row_v, [xg])
            ik = idx_v[0, pl.ds(g * L, L)]
            e = plsc.load_gather(e_tab_v, [ik])
            den = e
            num = e * e
            for k in range(1, N_POI_CAT):
                ik = idx_v[k, pl.ds(g * L, L)]
                e = plsc.load_gather(e_tab_v, [ik])
                den = den + e
                num = num + e * e
            p = num / den
            out_rows_v[dd, pl.ds(g * L, L)] = lv * ALPHA + p * (1.0 - ALPHA)
            return carry

        lax.fori_loop(0, NGRP, _phase_b, 0, unroll=2)

    d0 = (c * NS + s) * D_PER_TILE
    pltpu.sync_copy(out_rows_v, out_t_hbm.at[pl.ds(d0, D_PER_TILE)])


@jax.jit
def kernel(x, poi, emb_poi_weight, emb_loc_weight):
    run = pl.kernel(
        _body,
        out_type=jax.ShapeDtypeStruct((LOC_EMB_SIZE, BATCH), jnp.float32),
        mesh=plsc.VectorSubcoreMesh(core_axis_name="c", subcore_axis_name="s"),
        compiler_params=pltpu.CompilerParams(needs_layout_passes=False,
                                             use_tc_tiling_on_sc=True),
        scratch_types=[
            pltpu.VMEM((BATCH,), jnp.int32),
            pltpu.VMEM((POINT,), jnp.float32),
            pltpu.VMEM((N_POI_CAT, BATCH), jnp.int32),
            pltpu.VMEM((N_VALS, LOC_EMB_SIZE), jnp.float32),
            pltpu.VMEM((L,), jnp.float32),
            pltpu.VMEM((D_PER_TILE, BATCH), jnp.float32),
            pltpu.VMEM_SHARED((N_POI_CAT, BATCH), jnp.int32),
        ],
    )
    out_t = run(x.astype(jnp.int32), poi.T, emb_poi_weight, emb_loc_weight.T)
    return out_t.T


# speedup vs baseline: 4.1854x; 1.1574x over previous
import jax
import jax.numpy as jnp
from jax import lax
from jax.experimental import pallas as pl
from jax.experimental.pallas import tpu as pltpu
from jax.experimental.pallas import tpu_sc as plsc

LOC_EMB_SIZE = 64
N_POI_CAT = 16
N_VALS = 11
BATCH = 1024
POINT = 100000
ALPHA = 0.9
L = 16
NC, NS = 2, 16
D_PER_TILE = LOC_EMB_SIZE // (NC * NS)
NGRP = BATCH // L
SPLIT = 50048


def _body(x_hbm, poi_t_hbm, w_hbm, loc_t_hbm, out_t_hbm,
          x_v, row_v, idx_v, w_v, e0_v, e1_v, out_rows_v,
          sem_a, sem_b, sem_w, gath_sh):
    c = lax.axis_index("c")
    s = lax.axis_index("s")
    d0 = (c * NS + s) * D_PER_TILE

    cp_x = pltpu.async_copy(x_hbm, x_v, sem_a)
    cp_w = pltpu.async_copy(w_hbm, w_v, sem_w)
    cp_poi = pltpu.async_copy(poi_t_hbm.at[s], row_v, sem_b)
    cp_x.wait()
    cp_poi.wait()

    def _phase_a(g, carry):
        xg = x_v[pl.ds(g * L, L)]
        val = plsc.load_gather(row_v, [xg])
        idx_v[s, pl.ds(g * L, L)] = val.astype(jnp.int32)
        return carry

    lax.fori_loop(0, NGRP, _phase_a, 0, unroll=4)
    pltpu.sync_copy(idx_v.at[s], gath_sh.at[s])

    cp_u0 = pltpu.async_copy(loc_t_hbm.at[d0], row_v, sem_a)

    plsc.subcore_barrier()
    pltpu.sync_copy(gath_sh, idx_v)

    cp_w.wait()
    lane = lax.iota(jnp.int32, L)
    vlane = jnp.minimum(lane, N_VALS - 1)
    e_vec0 = jnp.exp(plsc.load_gather(w_v, [vlane, jnp.broadcast_to(d0, (L,))]))
    e_vec1 = jnp.exp(plsc.load_gather(w_v, [vlane, jnp.broadcast_to(d0 + 1, (L,))]))
    e0_v[...] = e_vec0
    e1_v[...] = e_vec1

    def _lv_pass(dd):
        def _p(g, carry):
            xg = x_v[pl.ds(g * L, L)]
            lv = plsc.load_gather(row_v, [xg])
            out_rows_v[dd, pl.ds(g * L, L)] = lv * ALPHA
            return carry
        lax.fori_loop(0, NGRP, _p, 0, unroll=4)

    cp_u0.wait()
    _lv_pass(0)
    cp_u1 = pltpu.async_copy(loc_t_hbm.at[d0 + 1], row_v, sem_b)

    def _p_pass(g, carry):
        xg = x_v[pl.ds(g * L, L)]
        ik = idx_v[0, pl.ds(g * L, L)]
        ga0 = plsc.load_gather(e0_v, [ik])
        ga1 = plsc.load_gather(e1_v, [ik])
        den0, num0 = ga0, ga0 * ga0
        den1, num1 = ga1, ga1 * ga1
        for k in range(1, N_POI_CAT):
            ik = idx_v[k, pl.ds(g * L, L)]
            ga0 = plsc.load_gather(e0_v, [ik])
            ga1 = plsc.load_gather(e1_v, [ik])
            den0 = den0 + ga0
            num0 = num0 + ga0 * ga0
            den1 = den1 + ga1
            num1 = num1 + ga1 * ga1
        o = out_rows_v[0, pl.ds(g * L, L)]
        out_rows_v[0, pl.ds(g * L, L)] = o + (num0 / den0) * (1.0 - ALPHA)
        out_rows_v[1, pl.ds(g * L, L)] = (num1 / den1) * (1.0 - ALPHA)
        return carry

    lax.fori_loop(0, NGRP, _p_pass, 0, unroll=2)

    cp_u1.wait()

    def _lv_add_pass(g, carry):
        xg = x_v[pl.ds(g * L, L)]
        lv = plsc.load_gather(row_v, [xg])
        o = out_rows_v[1, pl.ds(g * L, L)]
        out_rows_v[1, pl.ds(g * L, L)] = o + lv * ALPHA
        return carry

    lax.fori_loop(0, NGRP, _lv_add_pass, 0, unroll=4)

    pltpu.sync_copy(out_rows_v, out_t_hbm.at[pl.ds(d0, D_PER_TILE)])


@jax.jit
def kernel(x, poi, emb_poi_weight, emb_loc_weight):
    run = pl.kernel(
        _body,
        out_type=jax.ShapeDtypeStruct((LOC_EMB_SIZE, BATCH), jnp.float32),
        mesh=plsc.VectorSubcoreMesh(core_axis_name="c", subcore_axis_name="s"),
        compiler_params=pltpu.CompilerParams(needs_layout_passes=False,
                                             use_tc_tiling_on_sc=True),
        scratch_types=[
            pltpu.VMEM((BATCH,), jnp.int32),
            pltpu.VMEM((POINT,), jnp.float32),
            pltpu.VMEM((N_POI_CAT, BATCH), jnp.int32),
            pltpu.VMEM((N_VALS, LOC_EMB_SIZE), jnp.float32),
            pltpu.VMEM((L,), jnp.float32),
            pltpu.VMEM((L,), jnp.float32),
            pltpu.VMEM((D_PER_TILE, BATCH), jnp.float32),
            pltpu.SemaphoreType.DMA,
            pltpu.SemaphoreType.DMA,
            pltpu.SemaphoreType.DMA,
            pltpu.VMEM_SHARED((N_POI_CAT, BATCH), jnp.int32),
        ],
    )
    out_t = run(x.astype(jnp.int32), poi.T, emb_poi_weight, emb_loc_weight.T)
    return out_t.T
